# baseline (device time: 28607 ns/iter reference)
import jax
import jax.numpy as jnp
from jax import lax
from jax.experimental import pallas as pl
from jax.experimental.pallas import tpu as pltpu


def kernel(x, dest):
    m, n = x.shape
    xb = x.astype(jnp.bfloat16)
    dest2d = dest.reshape(8, -1)

    def body(xb_ref, dest_ref, rx_ref, rd_ref, send_sems, recv_sems):
        my_x = lax.axis_index("x")
        my_y = lax.axis_index("y")
        my_z = lax.axis_index("z")
        peer = (my_x, 1 - my_y, my_z)

        barrier = pltpu.get_barrier_semaphore()
        pl.semaphore_signal(
            barrier, inc=1, device_id=peer, device_id_type=pl.DeviceIdType.MESH
        )
        pl.semaphore_wait(barrier, 1)

        rdma_x = pltpu.make_async_remote_copy(
            src_ref=xb_ref,
            dst_ref=rx_ref,
            send_sem=send_sems.at[0],
            recv_sem=recv_sems.at[0],
            device_id=peer,
            device_id_type=pl.DeviceIdType.MESH,
        )
        rdma_d = pltpu.make_async_remote_copy(
            src_ref=dest_ref,
            dst_ref=rd_ref,
            send_sem=send_sems.at[1],
            recv_sem=recv_sems.at[1],
            device_id=peer,
            device_id_type=pl.DeviceIdType.MESH,
        )
        rdma_x.start()
        rdma_d.start()
        rdma_x.wait()
        rdma_d.wait()

    rx, rd = pl.pallas_call(
        body,
        out_shape=(
            jax.ShapeDtypeStruct((m, n), jnp.bfloat16),
            jax.ShapeDtypeStruct(dest2d.shape, jnp.int32),
        ),
        in_specs=[
            pl.BlockSpec(memory_space=pltpu.VMEM),
            pl.BlockSpec(memory_space=pltpu.VMEM),
        ],
        out_specs=(
            pl.BlockSpec(memory_space=pltpu.VMEM),
            pl.BlockSpec(memory_space=pltpu.VMEM),
        ),
        scratch_shapes=[
            pltpu.SemaphoreType.DMA((2,)),
            pltpu.SemaphoreType.DMA((2,)),
        ],
        compiler_params=pltpu.CompilerParams(collective_id=0),
    )(xb, dest2d)

    my_y = lax.axis_index("y")
    rdest = rd.reshape(-1)
    is_y0 = (my_y == 0)
    first_dest = jnp.where(is_y0, dest, rdest)
    second_dest = jnp.where(is_y0, rdest, dest)
    first_x = jnp.where(is_y0, xb, rx)
    second_x = jnp.where(is_y0, rx, xb)
    all_dest = jnp.concatenate([first_dest, second_dest])
    all_x = jnp.concatenate([first_x, second_x], axis=0)
    idx = jnp.argsort(all_dest != my_y, stable=True)[:m]
    return all_x[idx].astype(jnp.float32)
